# baseline (speedup 1.0000x reference)
import jax
import jax.numpy as jnp
from jax import lax
from jax.experimental import pallas as pl
from jax.experimental.pallas import tpu as pltpu
from jax.experimental.pallas import tpu_sc as plsc

B = 4
N = 8192
D_PATCH = 1024
HID = 256
K_SAMPLE = 512
KNN = 8
NT = 8
TN = N // NT


def _scores_kernel(img_ref, txt_ref, wi_ref, bi_ref, wt_ref, bt_ref,
                   out_ref, te_s):
    j = pl.program_id(0)

    @pl.when(j == 0)
    def _():
        te = jnp.dot(txt_ref[...], wt_ref[...],
                     preferred_element_type=jnp.float32) + bt_ref[...]
        nrm = jnp.sqrt(jnp.sum(te * te)) + 1e-12
        te_s[...] = te / nrm

    e = jnp.dot(img_ref[...], wi_ref[...],
                preferred_element_type=jnp.float32) + bi_ref[...]
    d = lax.dot_general(e, te_s[...], (((1,), (1,)), ((), ())),
                        preferred_element_type=jnp.float32)[:, 0]
    nrow = jnp.sqrt(jnp.sum(e * e, axis=1)) + 1e-12
    out_ref[0, :] = d / nrow


def _scores_slide(img_b, txt_b, w_img, b_img, w_txt, b_txt):
    return pl.pallas_call(
        _scores_kernel,
        grid=(NT,),
        in_specs=[
            pl.BlockSpec((TN, D_PATCH), lambda j: (j, 0)),
            pl.BlockSpec((1, D_PATCH), lambda j: (0, 0)),
            pl.BlockSpec((D_PATCH, HID), lambda j: (0, 0)),
            pl.BlockSpec((1, HID), lambda j: (0, 0)),
            pl.BlockSpec((D_PATCH, HID), lambda j: (0, 0)),
            pl.BlockSpec((1, HID), lambda j: (0, 0)),
        ],
        out_specs=pl.BlockSpec((1, TN), lambda j: (0, j)),
        out_shape=jax.ShapeDtypeStruct((1, N), jnp.float32),
        scratch_shapes=[pltpu.VMEM((1, HID), jnp.float32)],
    )(img_b, txt_b, w_img, b_img, w_txt, b_txt)


def _thresh_kernel(s_ref, out_ref):
    bits = lax.bitcast_convert_type(s_ref[...], jnp.int32)
    key = bits ^ ((bits >> 31) & jnp.int32(0x7FFFFFFF))

    def step(i, lo):
        t = lo + (jnp.int32(1) << (jnp.int32(31) - i))
        cnt = jnp.sum((key >= t).astype(jnp.int32), axis=1, keepdims=True)
        return jnp.where(cnt >= K_SAMPLE, t, lo)

    lo = jnp.full((1, 1), jnp.int32(-2**31))
    t = lax.fori_loop(0, 32, step, lo)
    tf = lax.bitcast_convert_type(
        jnp.where(t >= 0, t, t ^ jnp.int32(0x7FFFFFFF)), jnp.float32)
    out_ref[...] = jnp.broadcast_to(tf, (1, 16))


def _thresh_slide(scores_b):
    return pl.pallas_call(
        _thresh_kernel,
        grid=(),
        in_specs=[pl.BlockSpec((1, N), lambda: (0, 0))],
        out_specs=pl.BlockSpec((1, 16), lambda: (0, 0)),
        out_shape=jax.ShapeDtypeStruct((1, 16), jnp.float32),
    )(scores_b)


_GATHER_TILES = 32
_ROWS_PER = K_SAMPLE // _GATHER_TILES


def _make_sc_slide_body(b):
    def body(scores_h, thr_h, coords_h, img_h,
             sp_h, spt_h, sx_h,
             scores_v, thr_v, cx_v, cy_v,
             idx_v, spx_v, spy_v, spi_v,
             shared_idx, gidx_v, rows_v, sem):
        cid = lax.axis_index("c")
        sid = lax.axis_index("s")

        @pl.when(sid == 0)
        def _():
            pltpu.sync_copy(scores_h, scores_v)
            pltpu.sync_copy(thr_h, thr_v)
            pltpu.sync_copy(coords_h.at[pl.ds(0, N)], cx_v)
            pltpu.sync_copy(coords_h.at[pl.ds(N, N)], cy_v)
            thr = thr_v[...]

            def step(v, base):
                s = scores_v[pl.ds(v * 16, 16)]
                m = s >= thr
                mi = m.astype(jnp.int32)
                pos = base + plsc.cumsum(mi) - 1
                valid = jnp.logical_and(m, pos < K_SAMPLE)
                lane = lax.iota(jnp.int32, 16) + v * 16
                plsc.store_scatter(idx_v, [pos], lane, mask=valid)
                return base + jnp.sum(mi)

            lax.fori_loop(0, N // 16, step, jnp.int32(0))

            def step2(v, carry):
                iv = idx_v[pl.ds(v * 16, 16)]
                gx = plsc.load_gather(cx_v, [iv])
                gy = plsc.load_gather(cy_v, [iv])
                spx_v[pl.ds(v * 16, 16)] = gx
                spy_v[pl.ds(v * 16, 16)] = gy
                two = (lax.iota(jnp.int32, 16) + v * 16) * 2
                plsc.store_scatter(spi_v, [two], gx)
                plsc.store_scatter(spi_v, [two + 1], gy)
                idx_v[pl.ds(v * 16, 16)] = iv + b * N
                return carry

            lax.fori_loop(0, K_SAMPLE // 16, step2, 0)
            pltpu.sync_copy(idx_v, shared_idx)

            @pl.when(cid == 0)
            def _():
                pltpu.sync_copy(spx_v, sp_h.at[pl.ds(0, K_SAMPLE)])
                pltpu.sync_copy(spy_v, sp_h.at[pl.ds(K_SAMPLE, K_SAMPLE)])
                pltpu.sync_copy(spi_v, spt_h)

        plsc.subcore_barrier()

        wid = sid * 2 + cid
        base = wid * _ROWS_PER
        pltpu.sync_copy(shared_idx.at[pl.ds(base, _ROWS_PER)], gidx_v)
        pltpu.async_copy(img_h.at[gidx_v], rows_v, sem).wait()
        pltpu.sync_copy(rows_v, sx_h.at[pl.ds(base, _ROWS_PER)])

    return body


def _sc_slide(b, scores_flat, thr_flat, coords_flat, img2d):
    mesh = plsc.VectorSubcoreMesh(core_axis_name="c", subcore_axis_name="s")
    fn = pl.kernel(
        _make_sc_slide_body(b),
        out_type=(
            jax.ShapeDtypeStruct((2 * K_SAMPLE,), jnp.float32),
            jax.ShapeDtypeStruct((2 * K_SAMPLE,), jnp.float32),
            jax.ShapeDtypeStruct((K_SAMPLE, D_PATCH), jnp.float32),
        ),
        mesh=mesh,
        compiler_params=pltpu.CompilerParams(needs_layout_passes=False),
        scratch_types=[
            pltpu.VMEM((N,), jnp.float32),
            pltpu.VMEM((16,), jnp.float32),
            pltpu.VMEM((N,), jnp.float32),
            pltpu.VMEM((N,), jnp.float32),
            pltpu.VMEM((K_SAMPLE,), jnp.int32),
            pltpu.VMEM((K_SAMPLE,), jnp.float32),
            pltpu.VMEM((K_SAMPLE,), jnp.float32),
            pltpu.VMEM((2 * K_SAMPLE,), jnp.float32),
            pltpu.VMEM_SHARED((K_SAMPLE,), jnp.int32),
            pltpu.VMEM((_ROWS_PER,), jnp.int32),
            pltpu.VMEM((_ROWS_PER, D_PATCH), jnp.float32),
            pltpu.SemaphoreType.DMA,
        ],
    )
    return fn(scores_flat, thr_flat, coords_flat, img2d)


def _gin_kernel(sx0, sx1, sx2, sx3, sp0, sp1, sp2_, sp3, spt0, spt1,
                spt2, spt3,
                w1a_ref, b1a_ref, w1b_ref, b1b_ref,
                w2a_ref, b2a_ref, w2b_ref, b2b_ref,
                w3a_ref, b3a_ref, w3b_ref, b3b_ref,
                wa_ref, ba_ref, wb_ref, bb_ref, wc_ref,
                wr_ref, br_ref, wcls_ref, bcls_ref,
                out_ref):
    M = B * K_SAMPLE
    sx_refs = (sx0, sx1, sx2, sx3)
    sp_refs = (sp0, sp1, sp2_, sp3)
    spt_refs = (spt0, spt1, spt2, spt3)
    d2s = []
    for b in range(B):
        xrow = sp_refs[b][0:1, :]
        yrow = sp_refs[b][1:2, :]
        xcol = spt_refs[b][:, 0:1]
        ycol = spt_refs[b][:, 1:2]
        dx = xcol - xrow
        dy = ycol - yrow
        d2s.append(dx * dx + dy * dy)
    d2 = jnp.concatenate(d2s, axis=0)
    ii = lax.broadcasted_iota(jnp.int32, (M, K_SAMPLE), 0) % K_SAMPLE
    jj = lax.broadcasted_iota(jnp.int32, (M, K_SAMPLE), 1)
    d2 = jnp.where(ii == jj, jnp.float32(1e12), d2)

    adj = jnp.zeros((M, K_SAMPLE), jnp.float32)
    cur = d2
    for _ in range(KNN):
        m = jnp.min(cur, axis=1, keepdims=True)
        first_j = jnp.min(jnp.where(cur == m, jj, jnp.int32(2**30)),
                          axis=1, keepdims=True)
        sel = jj == first_j
        adj = jnp.where(sel, 1.0, adj)
        cur = jnp.where(sel, jnp.float32(1e30), cur)

    def agg(h):
        return jnp.concatenate(
            [jnp.dot(adj[b * K_SAMPLE:(b + 1) * K_SAMPLE],
                     h[b * K_SAMPLE:(b + 1) * K_SAMPLE],
                     preferred_element_type=jnp.float32)
             for b in range(B)], axis=0)

    y = jnp.concatenate(
        [jnp.dot(r[...], w1a_ref[...], preferred_element_type=jnp.float32)
         for r in sx_refs], axis=0)
    t1 = jax.nn.relu(y + agg(y) + b1a_ref[...])
    x1 = jax.nn.relu(
        jnp.dot(t1, w1b_ref[...], preferred_element_type=jnp.float32)
        + b1b_ref[...])

    t2 = jax.nn.relu(
        jnp.dot(x1 + agg(x1), w2a_ref[...],
                preferred_element_type=jnp.float32) + b2a_ref[...])
    x2 = jax.nn.relu(
        jnp.dot(t2, w2b_ref[...], preferred_element_type=jnp.float32)
        + b2b_ref[...])

    t3 = jax.nn.relu(
        jnp.dot(x2 + agg(x2), w3a_ref[...],
                preferred_element_type=jnp.float32) + b3a_ref[...])
    x3 = jax.nn.relu(
        jnp.dot(t3, w3b_ref[...], preferred_element_type=jnp.float32)
        + b3b_ref[...])

    a = jnp.tanh(
        jnp.dot(x3, wa_ref[...], preferred_element_type=jnp.float32)
        + ba_ref[...])
    g = jax.nn.sigmoid(
        jnp.dot(x3, wb_ref[...], preferred_element_type=jnp.float32)
        + bb_ref[...])
    att = jnp.sum(a * g * wc_ref[...], axis=1)[None, :]

    mx = jnp.max(att)
    e = jnp.exp(att - mx)
    p = e / jnp.sum(e)
    hp = jnp.dot(p, x3, preferred_element_type=jnp.float32)
    h = jax.nn.relu(
        jnp.dot(hp, wr_ref[...], preferred_element_type=jnp.float32)
        + br_ref[...])
    out_ref[...] = (jnp.dot(h, wcls_ref[...],
                            preferred_element_type=jnp.float32)
                    + bcls_ref[...])


def _gin(sxs, sps, spts, w1a, b1a, w1b, b1b, w2a, b2a, w2b, b2b,
         w3a, b3a, w3b, b3b, wa, ba, wb, bb, wc_row,
         w_rho, b_rho, w_cls, b_cls):
    full = lambda shape: pl.BlockSpec(shape, lambda: (0,) * len(shape))
    return pl.pallas_call(
        _gin_kernel,
        grid=(),
        in_specs=[
            *[full((K_SAMPLE, D_PATCH)) for _ in range(B)],
            *[full((2, K_SAMPLE)) for _ in range(B)],
            *[full((K_SAMPLE, 2)) for _ in range(B)],
            full((D_PATCH, HID)), full((1, HID)),
            full((HID, HID)), full((1, HID)),
            full((HID, HID)), full((1, HID)),
            full((HID, HID)), full((1, HID)),
            full((HID, HID)), full((1, HID)),
            full((HID, HID)), full((1, HID)),
            full((HID, HID)), full((1, HID)),
            full((HID, HID)), full((1, HID)),
            full((1, HID)),
            full((HID, HID)), full((1, HID)),
            full((HID, 4)), full((1, 4)),
        ],
        out_specs=pl.BlockSpec((1, 4), lambda: (0, 0)),
        out_shape=jax.ShapeDtypeStruct((1, 4), jnp.float32),
    )(*sxs, *sps, *spts, w1a, b1a, w1b, b1b, w2a, b2a, w2b, b2b,
      w3a, b3a, w3b, b3b, wa, ba, wb, bb, wc_row,
      w_rho, b_rho, w_cls, b_cls)


def kernel(image_patch_features_batch, original_patch_coordinates_batch,
           text_feat_batch,
           W_img, b_img, W_txt, b_txt,
           W1a, b1a, W1b, b1b, W2a, b2a, W2b, b2b, W3a, b3a, W3b, b3b,
           Wa_att, ba_att, Wb_att, bb_att, Wc_att, bc_att,
           W_rho, b_rho, W_cls, b_cls):
    del bc_att
    img = image_patch_features_batch
    img2d = img.reshape(B * N, D_PATCH)
    coords_t = jnp.transpose(original_patch_coordinates_batch, (0, 2, 1))
    b_img2 = b_img.reshape(1, HID)
    b_txt2 = b_txt.reshape(1, HID)
    sxs, sps, spts = [], [], []
    for b in range(B):
        scores_b = _scores_slide(img[b], text_feat_batch[b:b + 1],
                                 W_img, b_img2, W_txt, b_txt2)
        thr_b = _thresh_slide(scores_b)
        sp_b, spt_b, sx_b = _sc_slide(
            b, scores_b.reshape(-1), thr_b.reshape(-1),
            coords_t[b].reshape(-1), img2d)
        sxs.append(sx_b)
        sps.append(sp_b.reshape(2, K_SAMPLE))
        spts.append(spt_b.reshape(K_SAMPLE, 2))
    return _gin(
        sxs, sps, spts,
        W1a, b1a.reshape(1, HID), W1b, b1b.reshape(1, HID),
        W2a, b2a.reshape(1, HID), W2b, b2b.reshape(1, HID),
        W3a, b3a.reshape(1, HID), W3b, b3b.reshape(1, HID),
        Wa_att, ba_att.reshape(1, HID), Wb_att, bb_att.reshape(1, HID),
        Wc_att.reshape(1, HID),
        W_rho, b_rho.reshape(1, HID), W_cls, b_cls.reshape(1, 4))


# speedup vs baseline: 1.6531x; 1.6531x over previous
import jax
import jax.numpy as jnp
from jax import lax
from jax.experimental import pallas as pl
from jax.experimental.pallas import tpu as pltpu
from jax.experimental.pallas import tpu_sc as plsc

B = 4
N = 8192
D_PATCH = 1024
HID = 256
K_SAMPLE = 512
KNN = 8
NT = 8
TN = N // NT


def _scores_kernel(img_ref, txt_ref, wi_ref, bi_ref, wt_ref, bt_ref,
                   out_ref, te_s):
    b = pl.program_id(0)
    j = pl.program_id(1)

    @pl.when(jnp.logical_and(b == 0, j == 0))
    def _():
        te_t = lax.dot_general(wt_ref[...], txt_ref[...],
                               (((0,), (1,)), ((), ())),
                               preferred_element_type=jnp.float32)
        te_t = te_t + bt_ref[...].reshape(HID, 1)
        nrm = jnp.sqrt(jnp.sum(te_t * te_t, axis=0, keepdims=True)) + 1e-12
        te_s[...] = te_t / nrm

    e = jnp.dot(img_ref[0], wi_ref[...],
                preferred_element_type=jnp.float32) + bi_ref[...]
    d4 = jnp.dot(e, te_s[...], preferred_element_type=jnp.float32)
    sel = lax.broadcasted_iota(jnp.int32, (1, B), 1) == b
    d = jnp.sum(jnp.where(sel, d4, 0.0), axis=1)
    nrow = jnp.sqrt(jnp.sum(e * e, axis=1)) + 1e-12
    out_ref[0, 0, :] = d / nrow


def _scores(img, txt, w_img, b_img, w_txt, b_txt):
    return pl.pallas_call(
        _scores_kernel,
        grid=(B, NT),
        in_specs=[
            pl.BlockSpec((1, TN, D_PATCH), lambda b, j: (b, j, 0)),
            pl.BlockSpec((B, D_PATCH), lambda b, j: (0, 0)),
            pl.BlockSpec((D_PATCH, HID), lambda b, j: (0, 0)),
            pl.BlockSpec((1, HID), lambda b, j: (0, 0)),
            pl.BlockSpec((D_PATCH, HID), lambda b, j: (0, 0)),
            pl.BlockSpec((1, HID), lambda b, j: (0, 0)),
        ],
        out_specs=pl.BlockSpec((1, 1, TN), lambda b, j: (b, 0, j)),
        out_shape=jax.ShapeDtypeStruct((B, 1, N), jnp.float32),
        scratch_shapes=[pltpu.VMEM((HID, B), jnp.float32)],
    )(img, txt, w_img, b_img, w_txt, b_txt)


def _thresh_kernel(s_ref, out_ref):
    bits = lax.bitcast_convert_type(s_ref[:, 0, :], jnp.int32)
    key = bits ^ ((bits >> 31) & jnp.int32(0x7FFFFFFF))

    def step(i, lo):
        t = lo + (jnp.int32(1) << (jnp.int32(31) - i))
        cnt = jnp.sum((key >= t).astype(jnp.int32), axis=1, keepdims=True)
        return jnp.where(cnt >= K_SAMPLE, t, lo)

    lo = jnp.full((B, 1), jnp.int32(-2**31))
    t = lax.fori_loop(0, 32, step, lo)
    tf = lax.bitcast_convert_type(
        jnp.where(t >= 0, t, t ^ jnp.int32(0x7FFFFFFF)), jnp.float32)
    out_ref[...] = jnp.broadcast_to(tf[:, :, None], (B, 1, 16))


def _thresholds(scores):
    return pl.pallas_call(
        _thresh_kernel,
        grid=(1,),
        in_specs=[pl.BlockSpec((B, 1, N), lambda i: (0, 0, 0))],
        out_specs=pl.BlockSpec((B, 1, 16), lambda i: (0, 0, 0)),
        out_shape=jax.ShapeDtypeStruct((B, 1, 16), jnp.float32),
    )(scores)


def _sc_compact_body(scores_h, thr_h, coords_h, img_h,
                     sp_h, spt_h, sx_h,
                     scores_v, thr_v, cx_v, cy_v,
                     idx_v, spx_v, spy_v, spi_v,
                     shared_idx, gidx_v, rows_v, sem):
    cid = lax.axis_index("c")
    sid = lax.axis_index("s")

    @pl.when(sid < B)
    def _():
        b = sid
        pltpu.sync_copy(scores_h.at[pl.ds(b * N, N)], scores_v)
        pltpu.sync_copy(thr_h.at[pl.ds(b * 16, 16)], thr_v)
        pltpu.sync_copy(coords_h.at[pl.ds((b * 2 + 0) * N, N)], cx_v)
        pltpu.sync_copy(coords_h.at[pl.ds((b * 2 + 1) * N, N)], cy_v)
        thr = thr_v[...]

        def step(v, base):
            s = scores_v[pl.ds(v * 16, 16)]
            m = s >= thr
            mi = m.astype(jnp.int32)
            pos = base + plsc.cumsum(mi) - 1
            valid = jnp.logical_and(m, pos < K_SAMPLE)
            lane = lax.iota(jnp.int32, 16) + v * 16
            plsc.store_scatter(idx_v, [pos], lane, mask=valid)
            return base + jnp.sum(mi)

        lax.fori_loop(0, N // 16, step, jnp.int32(0))

        def step2(v, carry):
            iv = idx_v[pl.ds(v * 16, 16)]
            gx = plsc.load_gather(cx_v, [iv])
            gy = plsc.load_gather(cy_v, [iv])
            spx_v[pl.ds(v * 16, 16)] = gx
            spy_v[pl.ds(v * 16, 16)] = gy
            two = (lax.iota(jnp.int32, 16) + v * 16) * 2
            plsc.store_scatter(spi_v, [two], gx)
            plsc.store_scatter(spi_v, [two + 1], gy)
            idx_v[pl.ds(v * 16, 16)] = iv + b * N
            return carry

        lax.fori_loop(0, K_SAMPLE // 16, step2, 0)
        pltpu.sync_copy(idx_v, shared_idx.at[pl.ds(b * K_SAMPLE, K_SAMPLE)])

        @pl.when(cid == 0)
        def _():
            pltpu.sync_copy(
                spx_v, sp_h.at[pl.ds((b * 2 + 0) * K_SAMPLE, K_SAMPLE)])
            pltpu.sync_copy(
                spy_v, sp_h.at[pl.ds((b * 2 + 1) * K_SAMPLE, K_SAMPLE)])
            pltpu.sync_copy(
                spi_v, spt_h.at[pl.ds(b * 2 * K_SAMPLE, 2 * K_SAMPLE)])

    plsc.subcore_barrier()

    wid = sid * 2 + cid
    base = wid * _ROWS_PER
    pltpu.sync_copy(shared_idx.at[pl.ds(base, _ROWS_PER)], gidx_v)
    pltpu.async_copy(img_h.at[gidx_v], rows_v, sem).wait()
    pltpu.sync_copy(rows_v, sx_h.at[pl.ds(base, _ROWS_PER)])


_GATHER_TILES = 32
_ROWS_PER = B * K_SAMPLE // _GATHER_TILES


def _sc_select_gather(scores_flat, thr_flat, coords_flat, img2d):
    mesh = plsc.VectorSubcoreMesh(core_axis_name="c", subcore_axis_name="s")
    fn = pl.kernel(
        _sc_compact_body,
        out_type=(
            jax.ShapeDtypeStruct((B * 2 * K_SAMPLE,), jnp.float32),
            jax.ShapeDtypeStruct((B * 2 * K_SAMPLE,), jnp.float32),
            jax.ShapeDtypeStruct((B * K_SAMPLE, D_PATCH), jnp.float32),
        ),
        mesh=mesh,
        compiler_params=pltpu.CompilerParams(needs_layout_passes=False),
        scratch_types=[
            pltpu.VMEM((N,), jnp.float32),
            pltpu.VMEM((16,), jnp.float32),
            pltpu.VMEM((N,), jnp.float32),
            pltpu.VMEM((N,), jnp.float32),
            pltpu.VMEM((K_SAMPLE,), jnp.int32),
            pltpu.VMEM((K_SAMPLE,), jnp.float32),
            pltpu.VMEM((K_SAMPLE,), jnp.float32),
            pltpu.VMEM((2 * K_SAMPLE,), jnp.float32),
            pltpu.VMEM_SHARED((B * K_SAMPLE,), jnp.int32),
            pltpu.VMEM((_ROWS_PER,), jnp.int32),
            pltpu.VMEM((_ROWS_PER, D_PATCH), jnp.float32),
            pltpu.SemaphoreType.DMA,
        ],
    )
    return fn(scores_flat, thr_flat, coords_flat, img2d)


def _gin_kernel(sx_ref, sp_ref, spt_ref,
                w1a_ref, b1a_ref, w1b_ref, b1b_ref,
                w2a_ref, b2a_ref, w2b_ref, b2b_ref,
                w3a_ref, b3a_ref, w3b_ref, b3b_ref,
                wa_ref, ba_ref, wb_ref, bb_ref, wc_ref,
                wr_ref, br_ref, wcls_ref, bcls_ref,
                out_ref):
    M = B * K_SAMPLE
    d2s = []
    for b in range(B):
        xrow = sp_ref[b, 0:1, :]
        yrow = sp_ref[b, 1:2, :]
        xcol = spt_ref[b, :, 0:1]
        ycol = spt_ref[b, :, 1:2]
        dx = xcol - xrow
        dy = ycol - yrow
        d2s.append(dx * dx + dy * dy)
    d2 = jnp.concatenate(d2s, axis=0)
    ii = lax.broadcasted_iota(jnp.int32, (M, K_SAMPLE), 0) % K_SAMPLE
    jj = lax.broadcasted_iota(jnp.int32, (M, K_SAMPLE), 1)
    d2 = jnp.where(ii == jj, jnp.float32(1e12), d2)

    adj = jnp.zeros((M, K_SAMPLE), jnp.float32)
    cur = d2
    for _ in range(KNN):
        m = jnp.min(cur, axis=1, keepdims=True)
        first_j = jnp.min(jnp.where(cur == m, jj, jnp.int32(2**30)),
                          axis=1, keepdims=True)
        sel = jj == first_j
        adj = jnp.where(sel, 1.0, adj)
        cur = jnp.where(sel, jnp.float32(1e30), cur)

    def agg(h):
        return jnp.concatenate(
            [jnp.dot(adj[b * K_SAMPLE:(b + 1) * K_SAMPLE],
                     h[b * K_SAMPLE:(b + 1) * K_SAMPLE],
                     preferred_element_type=jnp.float32)
             for b in range(B)], axis=0)

    y = jnp.dot(sx_ref[...], w1a_ref[...], preferred_element_type=jnp.float32)
    t1 = jax.nn.relu(y + agg(y) + b1a_ref[...])
    x1 = jax.nn.relu(
        jnp.dot(t1, w1b_ref[...], preferred_element_type=jnp.float32)
        + b1b_ref[...])

    t2 = jax.nn.relu(
        jnp.dot(x1 + agg(x1), w2a_ref[...],
                preferred_element_type=jnp.float32) + b2a_ref[...])
    x2 = jax.nn.relu(
        jnp.dot(t2, w2b_ref[...], preferred_element_type=jnp.float32)
        + b2b_ref[...])

    t3 = jax.nn.relu(
        jnp.dot(x2 + agg(x2), w3a_ref[...],
                preferred_element_type=jnp.float32) + b3a_ref[...])
    x3 = jax.nn.relu(
        jnp.dot(t3, w3b_ref[...], preferred_element_type=jnp.float32)
        + b3b_ref[...])

    a = jnp.tanh(
        jnp.dot(x3, wa_ref[...], preferred_element_type=jnp.float32)
        + ba_ref[...])
    g = jax.nn.sigmoid(
        jnp.dot(x3, wb_ref[...], preferred_element_type=jnp.float32)
        + bb_ref[...])
    att = jnp.sum(a * g * wc_ref[...], axis=1)[None, :]

    mx = jnp.max(att)
    e = jnp.exp(att - mx)
    p = e / jnp.sum(e)
    hp = jnp.dot(p, x3, preferred_element_type=jnp.float32)
    h = jax.nn.relu(
        jnp.dot(hp, wr_ref[...], preferred_element_type=jnp.float32)
        + br_ref[...])
    out_ref[...] = (jnp.dot(h, wcls_ref[...],
                            preferred_element_type=jnp.float32)
                    + bcls_ref[...])


def _gin(sx, sp, spt, w1a, b1a, w1b, b1b, w2a, b2a, w2b, b2b,
         w3a, b3a, w3b, b3b, wa, ba, wb, bb, wc_row,
         w_rho, b_rho, w_cls, b_cls):
    full = lambda shape: pl.BlockSpec(shape, lambda: (0,) * len(shape))
    return pl.pallas_call(
        _gin_kernel,
        grid=(),
        in_specs=[
            full((B * K_SAMPLE, D_PATCH)),
            full((B, 2, K_SAMPLE)),
            full((B, K_SAMPLE, 2)),
            full((D_PATCH, HID)), full((1, HID)),
            full((HID, HID)), full((1, HID)),
            full((HID, HID)), full((1, HID)),
            full((HID, HID)), full((1, HID)),
            full((HID, HID)), full((1, HID)),
            full((HID, HID)), full((1, HID)),
            full((HID, HID)), full((1, HID)),
            full((HID, HID)), full((1, HID)),
            full((1, HID)),
            full((HID, HID)), full((1, HID)),
            full((HID, 4)), full((1, 4)),
        ],
        out_specs=pl.BlockSpec((1, 4), lambda: (0, 0)),
        out_shape=jax.ShapeDtypeStruct((1, 4), jnp.float32),
    )(sx, sp, spt, w1a, b1a, w1b, b1b, w2a, b2a, w2b, b2b,
      w3a, b3a, w3b, b3b, wa, ba, wb, bb, wc_row,
      w_rho, b_rho, w_cls, b_cls)


def kernel(image_patch_features_batch, original_patch_coordinates_batch,
           text_feat_batch,
           W_img, b_img, W_txt, b_txt,
           W1a, b1a, W1b, b1b, W2a, b2a, W2b, b2b, W3a, b3a, W3b, b3b,
           Wa_att, ba_att, Wb_att, bb_att, Wc_att, bc_att,
           W_rho, b_rho, W_cls, b_cls):
    del bc_att
    img = image_patch_features_batch
    scores = _scores(img, text_feat_batch, W_img, b_img.reshape(1, HID),
                     W_txt, b_txt.reshape(1, HID))
    thr = _thresholds(scores)
    coords_flat = jnp.transpose(
        original_patch_coordinates_batch, (0, 2, 1)).reshape(-1)
    sp_flat, spt_flat, sx = _sc_select_gather(
        scores.reshape(-1), thr.reshape(-1), coords_flat,
        img.reshape(B * N, D_PATCH))
    return _gin(
        sx,
        sp_flat.reshape(B, 2, K_SAMPLE),
        spt_flat.reshape(B, K_SAMPLE, 2),
        W1a, b1a.reshape(1, HID), W1b, b1b.reshape(1, HID),
        W2a, b2a.reshape(1, HID), W2b, b2b.reshape(1, HID),
        W3a, b3a.reshape(1, HID), W3b, b3b.reshape(1, HID),
        Wa_att, ba_att.reshape(1, HID), Wb_att, bb_att.reshape(1, HID),
        Wc_att.reshape(1, HID),
        W_rho, b_rho.reshape(1, HID), W_cls, b_cls.reshape(1, 4))
